# baseline (speedup 1.0000x reference)
import functools

import jax
import jax.numpy as jnp
import numpy as np
from jax import lax
from jax.experimental import pallas as pl
from jax.experimental.pallas import tpu as pltpu
from jax.experimental.pallas import tpu_sc as plsc

ATOM = 32
BOND = 16
N_NODES = 10000
N_EDGES = 160000

NC, NS = 2, 16
NW = NC * NS
CH = 128
NCHUNK = N_EDGES // CH
ROWS_PER_SUB = N_NODES // NS

_SC_MESH = plsc.VectorSubcoreMesh(
    core_axis_name="c", subcore_axis_name="s", num_cores=NC, num_subcores=NS)
_SC_MESH1 = plsc.VectorSubcoreMesh(
    core_axis_name="c", subcore_axis_name="s", num_cores=1, num_subcores=NS)


def _worker_range(wid, nw):
    cpw, rem = NCHUNK // nw, NCHUNK % nw
    maxc = cpw + 1
    c0 = cpw * wid + jnp.minimum(wid, rem)
    nloc = cpw + jnp.where(wid < rem, 1, 0)
    sbase = jnp.minimum(c0, NCHUNK - maxc)
    return c0, nloc, c0 - sbase, sbase


MAXC = NCHUNK // NW + 1
MAXC1 = NCHUNK // NS + 1


def _build_gather(interpret=False):
    @functools.partial(
        pl.kernel,
        out_type=jax.ShapeDtypeStruct((N_EDGES, ATOM), jnp.bfloat16),
        mesh=_SC_MESH,
        scratch_types=[
            pltpu.VMEM((MAXC, CH), jnp.int32),
            pltpu.VMEM((CH, ATOM), jnp.bfloat16),
            pltpu.VMEM((CH, ATOM), jnp.bfloat16),
            pltpu.SemaphoreType.DMA,
            pltpu.SemaphoreType.DMA,
        ],
        compiler_params=pltpu.CompilerParams(use_tc_tiling_on_sc=False, needs_layout_passes=False),
        interpret=interpret,
    )
    def _gather(table_hbm, pair_hbm, out_hbm, idx_all, rows0, rows1,
                sem0, sem1):
        wid = lax.axis_index("s") * NC + lax.axis_index("c")
        c0, nloc, off, sbase = _worker_range(wid, NW)
        pltpu.sync_copy(pair_hbm.at[1, pl.ds(sbase, MAXC)], idx_all)
        rows = (rows0, rows1)
        sems = (sem0, sem1)

        def body(i, carry):
            b = i % 2
            for p in range(2):
                @pl.when(b == p)
                def _():
                    pltpu.async_copy(
                        table_hbm.at[idx_all.at[off + i]], rows[p], sems[p])

            @pl.when(i >= 1)
            def _():
                for p in range(2):
                    @pl.when((1 - b) == p)
                    def _():
                        pltpu.make_async_copy(
                            table_hbm.at[idx_all.at[off + i - 1]],
                            rows[p], sems[p]).wait()
                        pltpu.sync_copy(
                            rows[p],
                            out_hbm.at[pl.ds((c0 + i - 1) * CH, CH)])
            return carry

        lax.fori_loop(0, nloc, body, 0)
        for p in range(2):
            @pl.when(((nloc - 1) % 2) == p)
            def _():
                pltpu.make_async_copy(
                    table_hbm.at[idx_all.at[off + nloc - 1]],
                    rows[p], sems[p]).wait()
                pltpu.sync_copy(
                    rows[p], out_hbm.at[pl.ds((c0 + nloc - 1) * CH, CH)])

    return _gather


_gather = _build_gather()


TB = 3200
KZ = BOND * ATOM


def _transform_body(bond_ref, neigh_ref, q_ref, wz_ref, bt_ref, out_ref):
    bond = bond_ref[...].astype(jnp.bfloat16)
    neigh = neigh_ref[...]
    rep = jnp.tile(bond, (ATOM, 1))
    til = jnp.dot(q_ref[...], neigh,
                  preferred_element_type=jnp.float32).astype(jnp.bfloat16)
    z = rep * til
    out_ref[...] = (
        jnp.dot(wz_ref[...], z, preferred_element_type=jnp.float32)
        + jnp.dot(bt_ref[...], neigh, preferred_element_type=jnp.float32)
    ).astype(jnp.bfloat16)


def _build_transform(interpret=False):
    return pl.pallas_call(
        _transform_body,
        grid=(N_EDGES // TB,),
        in_specs=[
            pl.BlockSpec((BOND, TB), lambda i: (0, i)),
            pl.BlockSpec((ATOM, TB), lambda i: (0, i)),
            pl.BlockSpec((KZ, ATOM), lambda i: (0, 0)),
            pl.BlockSpec((ATOM, KZ), lambda i: (0, 0)),
            pl.BlockSpec((ATOM, ATOM), lambda i: (0, 0)),
        ],
        out_specs=pl.BlockSpec((ATOM, TB), lambda i: (0, i)),
        out_shape=jax.ShapeDtypeStruct((ATOM, N_EDGES), jnp.bfloat16),
        interpret=interpret,
    )


_transform = _build_transform()


def _build_scatter(interpret=False):
    @functools.partial(
        pl.kernel,
        out_type=jax.ShapeDtypeStruct((N_NODES, ATOM), jnp.bfloat16),
        mesh=_SC_MESH1,
        scratch_types=[
            pltpu.VMEM((MAXC1, CH), jnp.int32),
            pltpu.VMEM((CH, ATOM), jnp.bfloat16),
            pltpu.VMEM((CH, ATOM), jnp.bfloat16),
            pltpu.VMEM_SHARED((N_NODES, ATOM), jnp.bfloat16),
            pltpu.SemaphoreType.DMA,
            pltpu.SemaphoreType.DMA,
        ],
        compiler_params=pltpu.CompilerParams(use_tc_tiling_on_sc=False, needs_layout_passes=False),
        interpret=interpret,
    )
    def _scatter(vals_hbm, pair_hbm, zeros_hbm, out_hbm, idx_all,
                 val0, val1, acc_sh, sem0, sem1):
        s = lax.axis_index("s")
        wid = s
        pltpu.sync_copy(
            zeros_hbm.at[pl.ds(s * ROWS_PER_SUB, ROWS_PER_SUB)],
            acc_sh.at[pl.ds(s * ROWS_PER_SUB, ROWS_PER_SUB)],
        )
        c0, nloc, off, sbase = _worker_range(wid, NS)
        pltpu.sync_copy(pair_hbm.at[0, pl.ds(sbase, MAXC1)], idx_all)
        plsc.subcore_barrier()
        vals = (val0, val1)
        sems = (sem0, sem1)

        def body(i, carry):
            b = i % 2
            for p in range(2):
                @pl.when(b == p)
                def _():
                    pltpu.async_copy(
                        vals_hbm.at[pl.ds((c0 + i) * CH, CH)],
                        vals[p], sems[p])

            @pl.when(i >= 1)
            def _():
                for p in range(2):
                    @pl.when((1 - b) == p)
                    def _():
                        pltpu.make_async_copy(
                            vals_hbm.at[pl.ds((c0 + i - 1) * CH, CH)],
                            vals[p], sems[p]).wait()
                        pltpu.sync_copy(
                            vals[p], acc_sh.at[idx_all.at[off + i - 1]],
                            add=True)
            return carry

        lax.fori_loop(0, nloc, body, 0)
        for p in range(2):
            @pl.when(((nloc - 1) % 2) == p)
            def _():
                pltpu.make_async_copy(
                    vals_hbm.at[pl.ds((c0 + nloc - 1) * CH, CH)],
                    vals[p], sems[p]).wait()
                pltpu.sync_copy(
                    vals[p], acc_sh.at[idx_all.at[off + nloc - 1]], add=True)
        plsc.subcore_barrier()
        pltpu.sync_copy(
            acc_sh.at[pl.ds(s * ROWS_PER_SUB, ROWS_PER_SUB)],
            out_hbm.at[pl.ds(s * ROWS_PER_SUB, ROWS_PER_SUB)],
        )

    return _scatter


_scatter = _build_scatter()


def kernel(atom_features, bond_features, pair_indices, kernel, bias):
    pair3 = pair_indices.astype(jnp.int32).T.reshape(2, NCHUNK, CH)

    qmat = jnp.asarray(np.kron(np.eye(ATOM, dtype=np.float32),
                               np.ones((BOND, 1), dtype=np.float32)),
                       dtype=jnp.bfloat16)
    wz = kernel.reshape(BOND, ATOM, ATOM).transpose(1, 2, 0).reshape(
        ATOM, KZ).astype(jnp.bfloat16)
    bt = bias.reshape(ATOM, ATOM).astype(jnp.bfloat16)

    neigh = _gather(atom_features.astype(jnp.bfloat16), pair3)
    transformed_t = _transform(bond_features.T, neigh.T, qmat, wz, bt)
    out = _scatter(transformed_t.T, pair3,
                   jnp.zeros((N_NODES, ATOM), jnp.bfloat16))
    return out.astype(jnp.float32)


# speedup vs baseline: 1.0746x; 1.0746x over previous
import functools

import jax
import jax.numpy as jnp
import numpy as np
from jax import lax
from jax.experimental import pallas as pl
from jax.experimental.pallas import tpu as pltpu
from jax.experimental.pallas import tpu_sc as plsc

ATOM = 32
BOND = 16
N_NODES = 10000
N_EDGES = 160000

NC, NS = 2, 16
NW = NC * NS
CH = 128
NCHUNK = N_EDGES // CH
ROWS_PER_SUB = N_NODES // NS

_SC_MESH = plsc.VectorSubcoreMesh(
    core_axis_name="c", subcore_axis_name="s", num_cores=NC, num_subcores=NS)
_SC_MESH1 = plsc.VectorSubcoreMesh(
    core_axis_name="c", subcore_axis_name="s", num_cores=1, num_subcores=NS)


def _worker_range(wid, nw):
    cpw, rem = NCHUNK // nw, NCHUNK % nw
    maxc = cpw + 1
    c0 = cpw * wid + jnp.minimum(wid, rem)
    nloc = cpw + jnp.where(wid < rem, 1, 0)
    sbase = jnp.minimum(c0, NCHUNK - maxc)
    return c0, nloc, c0 - sbase, sbase


MAXC = NCHUNK // NW + 1
MAXC1 = NCHUNK // NS + 1


def _build_gather(interpret=False):
    @functools.partial(
        pl.kernel,
        out_type=jax.ShapeDtypeStruct((N_EDGES, ATOM), jnp.bfloat16),
        mesh=_SC_MESH,
        scratch_types=[
            pltpu.VMEM((MAXC, CH), jnp.int32),
            pltpu.VMEM((CH, ATOM), jnp.bfloat16),
            pltpu.VMEM((CH, ATOM), jnp.bfloat16),
            pltpu.SemaphoreType.DMA,
            pltpu.SemaphoreType.DMA,
        ],
        compiler_params=pltpu.CompilerParams(use_tc_tiling_on_sc=False, needs_layout_passes=False),
        interpret=interpret,
    )
    def _gather(table_hbm, pair_hbm, out_hbm, idx_all, rows0, rows1,
                sem0, sem1):
        wid = lax.axis_index("s") * NC + lax.axis_index("c")
        c0, nloc, off, sbase = _worker_range(wid, NW)
        pltpu.sync_copy(pair_hbm.at[1, pl.ds(sbase, MAXC)], idx_all)
        rows = (rows0, rows1)
        sems = (sem0, sem1)

        def body(i, carry):
            b = i % 2
            for p in range(2):
                @pl.when(b == p)
                def _():
                    pltpu.async_copy(
                        table_hbm.at[idx_all.at[off + i]], rows[p], sems[p])

            @pl.when(i >= 1)
            def _():
                for p in range(2):
                    @pl.when((1 - b) == p)
                    def _():
                        pltpu.make_async_copy(
                            table_hbm.at[idx_all.at[off + i - 1]],
                            rows[p], sems[p]).wait()
                        pltpu.sync_copy(
                            rows[p],
                            out_hbm.at[pl.ds((c0 + i - 1) * CH, CH)])
            return carry

        lax.fori_loop(0, nloc, body, 0)
        for p in range(2):
            @pl.when(((nloc - 1) % 2) == p)
            def _():
                pltpu.make_async_copy(
                    table_hbm.at[idx_all.at[off + nloc - 1]],
                    rows[p], sems[p]).wait()
                pltpu.sync_copy(
                    rows[p], out_hbm.at[pl.ds((c0 + nloc - 1) * CH, CH)])

    return _gather


_gather = _build_gather()


TB = 3200
KZ = BOND * ATOM


def _transform_body(bond_ref, neigh_ref, q_ref, wz_ref, bt_ref, out_ref):
    bond = bond_ref[...].astype(jnp.bfloat16).T
    neigh = neigh_ref[...]
    rep = jnp.tile(bond, (1, ATOM))
    til = jnp.dot(neigh, q_ref[...],
                  preferred_element_type=jnp.float32).astype(jnp.bfloat16)
    z = rep * til
    out_ref[...] = (
        jnp.dot(z, wz_ref[...], preferred_element_type=jnp.float32)
        + jnp.dot(neigh, bt_ref[...], preferred_element_type=jnp.float32)
    ).astype(jnp.bfloat16)


def _build_transform(interpret=False):
    return pl.pallas_call(
        _transform_body,
        grid=(N_EDGES // TB,),
        in_specs=[
            pl.BlockSpec((BOND, TB), lambda i: (0, i)),
            pl.BlockSpec((TB, ATOM), lambda i: (i, 0)),
            pl.BlockSpec((ATOM, KZ), lambda i: (0, 0)),
            pl.BlockSpec((KZ, ATOM), lambda i: (0, 0)),
            pl.BlockSpec((ATOM, ATOM), lambda i: (0, 0)),
        ],
        out_specs=pl.BlockSpec((TB, ATOM), lambda i: (i, 0)),
        out_shape=jax.ShapeDtypeStruct((N_EDGES, ATOM), jnp.bfloat16),
        interpret=interpret,
    )


_transform = _build_transform()


def _build_scatter(interpret=False):
    @functools.partial(
        pl.kernel,
        out_type=jax.ShapeDtypeStruct((N_NODES, ATOM), jnp.bfloat16),
        mesh=_SC_MESH1,
        scratch_types=[
            pltpu.VMEM((MAXC1, CH), jnp.int32),
            pltpu.VMEM((CH, ATOM), jnp.bfloat16),
            pltpu.VMEM((CH, ATOM), jnp.bfloat16),
            pltpu.VMEM_SHARED((N_NODES, ATOM), jnp.bfloat16),
            pltpu.SemaphoreType.DMA,
            pltpu.SemaphoreType.DMA,
        ],
        compiler_params=pltpu.CompilerParams(use_tc_tiling_on_sc=False, needs_layout_passes=False),
        interpret=interpret,
    )
    def _scatter(vals_hbm, pair_hbm, zeros_hbm, out_hbm, idx_all,
                 val0, val1, acc_sh, sem0, sem1):
        s = lax.axis_index("s")
        wid = s
        pltpu.sync_copy(
            zeros_hbm.at[pl.ds(s * ROWS_PER_SUB, ROWS_PER_SUB)],
            acc_sh.at[pl.ds(s * ROWS_PER_SUB, ROWS_PER_SUB)],
        )
        c0, nloc, off, sbase = _worker_range(wid, NS)
        pltpu.sync_copy(pair_hbm.at[0, pl.ds(sbase, MAXC1)], idx_all)
        plsc.subcore_barrier()
        vals = (val0, val1)
        sems = (sem0, sem1)

        def body(i, carry):
            b = i % 2
            for p in range(2):
                @pl.when(b == p)
                def _():
                    pltpu.async_copy(
                        vals_hbm.at[pl.ds((c0 + i) * CH, CH)],
                        vals[p], sems[p])

            @pl.when(i >= 1)
            def _():
                for p in range(2):
                    @pl.when((1 - b) == p)
                    def _():
                        pltpu.make_async_copy(
                            vals_hbm.at[pl.ds((c0 + i - 1) * CH, CH)],
                            vals[p], sems[p]).wait()
                        pltpu.sync_copy(
                            vals[p], acc_sh.at[idx_all.at[off + i - 1]],
                            add=True)
            return carry

        lax.fori_loop(0, nloc, body, 0)
        for p in range(2):
            @pl.when(((nloc - 1) % 2) == p)
            def _():
                pltpu.make_async_copy(
                    vals_hbm.at[pl.ds((c0 + nloc - 1) * CH, CH)],
                    vals[p], sems[p]).wait()
                pltpu.sync_copy(
                    vals[p], acc_sh.at[idx_all.at[off + nloc - 1]], add=True)
        plsc.subcore_barrier()
        pltpu.sync_copy(
            acc_sh.at[pl.ds(s * ROWS_PER_SUB, ROWS_PER_SUB)],
            out_hbm.at[pl.ds(s * ROWS_PER_SUB, ROWS_PER_SUB)],
        )

    return _scatter


_scatter = _build_scatter()


def kernel(atom_features, bond_features, pair_indices, kernel, bias):
    pair3 = pair_indices.astype(jnp.int32).T.reshape(2, NCHUNK, CH)

    qmat = jnp.asarray(np.kron(np.eye(ATOM, dtype=np.float32),
                               np.ones((1, BOND), dtype=np.float32)),
                       dtype=jnp.bfloat16)
    wz = kernel.reshape(BOND, ATOM, ATOM).transpose(2, 0, 1).reshape(
        KZ, ATOM).astype(jnp.bfloat16)
    bt = bias.reshape(ATOM, ATOM).T.astype(jnp.bfloat16)

    neigh = _gather(atom_features.astype(jnp.bfloat16), pair3)
    transformed = _transform(bond_features.T, neigh, qmat, wz, bt)
    out = _scatter(transformed, pair3,
                   jnp.zeros((N_NODES, ATOM), jnp.bfloat16))
    return out.astype(jnp.float32)


# speedup vs baseline: 1.0891x; 1.0134x over previous
import functools

import jax
import jax.numpy as jnp
import numpy as np
from jax import lax
from jax.experimental import pallas as pl
from jax.experimental.pallas import tpu as pltpu
from jax.experimental.pallas import tpu_sc as plsc

ATOM = 32
BOND = 16
N_NODES = 10000
N_EDGES = 160000

NC, NS = 2, 16
NW = NC * NS
CH = 128
NCHUNK = N_EDGES // CH
ROWS_PER_SUB = N_NODES // NS

_SC_MESH = plsc.VectorSubcoreMesh(
    core_axis_name="c", subcore_axis_name="s", num_cores=NC, num_subcores=NS)
_SC_MESH1 = plsc.VectorSubcoreMesh(
    core_axis_name="c", subcore_axis_name="s", num_cores=1, num_subcores=NS)


def _worker_range(wid, nw):
    cpw, rem = NCHUNK // nw, NCHUNK % nw
    maxc = cpw + 1
    c0 = cpw * wid + jnp.minimum(wid, rem)
    nloc = cpw + jnp.where(wid < rem, 1, 0)
    sbase = jnp.minimum(c0, NCHUNK - maxc)
    return c0, nloc, c0 - sbase, sbase


MAXC = NCHUNK // NW + 1
MAXC1 = NCHUNK // NS + 1


def _build_gather(interpret=False):
    @functools.partial(
        pl.kernel,
        out_type=jax.ShapeDtypeStruct((N_EDGES, ATOM), jnp.bfloat16),
        mesh=_SC_MESH,
        scratch_types=[
            pltpu.VMEM((MAXC, CH), jnp.int32),
            pltpu.VMEM((CH, ATOM), jnp.bfloat16),
            pltpu.VMEM((CH, ATOM), jnp.bfloat16),
            pltpu.SemaphoreType.DMA,
            pltpu.SemaphoreType.DMA,
        ],
        compiler_params=pltpu.CompilerParams(use_tc_tiling_on_sc=False, needs_layout_passes=False),
        interpret=interpret,
    )
    def _gather(table_hbm, pair_hbm, out_hbm, idx_all, rows0, rows1,
                sem0, sem1):
        wid = lax.axis_index("s") * NC + lax.axis_index("c")
        c0, nloc, off, sbase = _worker_range(wid, NW)
        pltpu.sync_copy(pair_hbm.at[1, pl.ds(sbase, MAXC)], idx_all)
        rows = (rows0, rows1)
        sems = (sem0, sem1)

        def body(i, carry):
            b = i % 2
            for p in range(2):
                @pl.when(b == p)
                def _():
                    pltpu.async_copy(
                        table_hbm.at[idx_all.at[off + i]], rows[p], sems[p])

            @pl.when(i >= 1)
            def _():
                for p in range(2):
                    @pl.when((1 - b) == p)
                    def _():
                        pltpu.make_async_copy(
                            table_hbm.at[idx_all.at[off + i - 1]],
                            rows[p], sems[p]).wait()
                        pltpu.sync_copy(
                            rows[p],
                            out_hbm.at[pl.ds((c0 + i - 1) * CH, CH)])
            return carry

        lax.fori_loop(0, nloc, body, 0)
        for p in range(2):
            @pl.when(((nloc - 1) % 2) == p)
            def _():
                pltpu.make_async_copy(
                    table_hbm.at[idx_all.at[off + nloc - 1]],
                    rows[p], sems[p]).wait()
                pltpu.sync_copy(
                    rows[p], out_hbm.at[pl.ds((c0 + nloc - 1) * CH, CH)])

    return _gather


_gather = _build_gather()


TB = 6400
KZ = BOND * ATOM


def _transform_body(bond_ref, neigh_ref, q_ref, wz_ref, bt_ref, out_ref):
    bond = bond_ref[...].astype(jnp.bfloat16).T
    neigh = neigh_ref[...]
    rep = jnp.tile(bond, (1, ATOM))
    til = jnp.dot(neigh, q_ref[...],
                  preferred_element_type=jnp.float32).astype(jnp.bfloat16)
    z = rep * til
    out_ref[...] = (
        jnp.dot(z, wz_ref[...], preferred_element_type=jnp.float32)
        + jnp.dot(neigh, bt_ref[...], preferred_element_type=jnp.float32)
    ).astype(jnp.bfloat16)


def _build_transform(interpret=False):
    return pl.pallas_call(
        _transform_body,
        grid=(N_EDGES // TB,),
        in_specs=[
            pl.BlockSpec((BOND, TB), lambda i: (0, i)),
            pl.BlockSpec((TB, ATOM), lambda i: (i, 0)),
            pl.BlockSpec((ATOM, KZ), lambda i: (0, 0)),
            pl.BlockSpec((KZ, ATOM), lambda i: (0, 0)),
            pl.BlockSpec((ATOM, ATOM), lambda i: (0, 0)),
        ],
        out_specs=pl.BlockSpec((TB, ATOM), lambda i: (i, 0)),
        out_shape=jax.ShapeDtypeStruct((N_EDGES, ATOM), jnp.bfloat16),
        interpret=interpret,
    )


_transform = _build_transform()


def _build_scatter(interpret=False):
    @functools.partial(
        pl.kernel,
        out_type=jax.ShapeDtypeStruct((N_NODES, ATOM), jnp.bfloat16),
        mesh=_SC_MESH1,
        scratch_types=[
            pltpu.VMEM((MAXC1, CH), jnp.int32),
            pltpu.VMEM((CH, ATOM), jnp.bfloat16),
            pltpu.VMEM((CH, ATOM), jnp.bfloat16),
            pltpu.VMEM_SHARED((N_NODES, ATOM), jnp.bfloat16),
            pltpu.SemaphoreType.DMA,
            pltpu.SemaphoreType.DMA,
        ],
        compiler_params=pltpu.CompilerParams(use_tc_tiling_on_sc=False, needs_layout_passes=False),
        interpret=interpret,
    )
    def _scatter(vals_hbm, pair_hbm, zeros_hbm, out_hbm, idx_all,
                 val0, val1, acc_sh, sem0, sem1):
        s = lax.axis_index("s")
        wid = s
        pltpu.sync_copy(
            zeros_hbm.at[pl.ds(s * ROWS_PER_SUB, ROWS_PER_SUB)],
            acc_sh.at[pl.ds(s * ROWS_PER_SUB, ROWS_PER_SUB)],
        )
        c0, nloc, off, sbase = _worker_range(wid, NS)
        pltpu.sync_copy(pair_hbm.at[0, pl.ds(sbase, MAXC1)], idx_all)
        plsc.subcore_barrier()
        vals = (val0, val1)
        sems = (sem0, sem1)

        def body(i, carry):
            b = i % 2
            for p in range(2):
                @pl.when(b == p)
                def _():
                    pltpu.async_copy(
                        vals_hbm.at[pl.ds((c0 + i) * CH, CH)],
                        vals[p], sems[p])

            @pl.when(i >= 1)
            def _():
                for p in range(2):
                    @pl.when((1 - b) == p)
                    def _():
                        pltpu.make_async_copy(
                            vals_hbm.at[pl.ds((c0 + i - 1) * CH, CH)],
                            vals[p], sems[p]).wait()
                        pltpu.sync_copy(
                            vals[p], acc_sh.at[idx_all.at[off + i - 1]],
                            add=True)
            return carry

        lax.fori_loop(0, nloc, body, 0)
        for p in range(2):
            @pl.when(((nloc - 1) % 2) == p)
            def _():
                pltpu.make_async_copy(
                    vals_hbm.at[pl.ds((c0 + nloc - 1) * CH, CH)],
                    vals[p], sems[p]).wait()
                pltpu.sync_copy(
                    vals[p], acc_sh.at[idx_all.at[off + nloc - 1]], add=True)
        plsc.subcore_barrier()
        pltpu.sync_copy(
            acc_sh.at[pl.ds(s * ROWS_PER_SUB, ROWS_PER_SUB)],
            out_hbm.at[pl.ds(s * ROWS_PER_SUB, ROWS_PER_SUB)],
        )

    return _scatter


_scatter = _build_scatter()


def kernel(atom_features, bond_features, pair_indices, kernel, bias):
    pair3 = pair_indices.astype(jnp.int32).T.reshape(2, NCHUNK, CH)

    qmat = jnp.asarray(np.kron(np.eye(ATOM, dtype=np.float32),
                               np.ones((1, BOND), dtype=np.float32)),
                       dtype=jnp.bfloat16)
    wz = kernel.reshape(BOND, ATOM, ATOM).transpose(2, 0, 1).reshape(
        KZ, ATOM).astype(jnp.bfloat16)
    bt = bias.reshape(ATOM, ATOM).T.astype(jnp.bfloat16)

    neigh = _gather(atom_features.astype(jnp.bfloat16), pair3)
    transformed = _transform(bond_features.T, neigh, qmat, wz, bt)
    out = _scatter(transformed, pair3,
                   jnp.zeros((N_NODES, ATOM), jnp.bfloat16))
    return out.astype(jnp.float32)
